# baseline (speedup 1.0000x reference)
import jax
import jax.numpy as jnp
from jax.experimental import pallas as pl
from jax.experimental.pallas import tpu as pltpu

_SEQ_BLK = 512


def _add_body(x_ref, pe_ref, o_ref):
    o_ref[...] = x_ref[...] + pe_ref[...]


def kernel(x, pe_weight):
    batch, seq, d = x.shape
    grid = (seq // _SEQ_BLK, batch)
    return pl.pallas_call(
        _add_body,
        grid=grid,
        in_specs=[
            pl.BlockSpec((1, _SEQ_BLK, d), lambda s, b: (b, s, 0)),
            pl.BlockSpec((_SEQ_BLK, d), lambda s, b: (s, 0)),
        ],
        out_specs=pl.BlockSpec((1, _SEQ_BLK, d), lambda s, b: (b, s, 0)),
        out_shape=jax.ShapeDtypeStruct((batch, seq, d), x.dtype),
        compiler_params=pltpu.CompilerParams(
            dimension_semantics=("arbitrary", "arbitrary"),
        ),
    )(x, pe_weight)


# speedup vs baseline: 1.1631x; 1.1631x over previous
import jax
import jax.numpy as jnp
from jax.experimental import pallas as pl
from jax.experimental.pallas import tpu as pltpu

_SEQ_BLK = 2048


def _add_body(x_ref, pe_ref, o_ref):
    o_ref[...] = x_ref[...] + pe_ref[...]


def kernel(x, pe_weight):
    batch, seq, d = x.shape
    grid = (seq // _SEQ_BLK, batch)
    return pl.pallas_call(
        _add_body,
        grid=grid,
        in_specs=[
            pl.BlockSpec((1, _SEQ_BLK, d), lambda s, b: (b, s, 0)),
            pl.BlockSpec((_SEQ_BLK, d), lambda s, b: (s, 0)),
        ],
        out_specs=pl.BlockSpec((1, _SEQ_BLK, d), lambda s, b: (b, s, 0)),
        out_shape=jax.ShapeDtypeStruct((batch, seq, d), x.dtype),
        compiler_params=pltpu.CompilerParams(
            dimension_semantics=("arbitrary", "arbitrary"),
        ),
    )(x, pe_weight)
